# baseline (speedup 1.0000x reference)
import functools

import jax
import jax.numpy as jnp
import numpy as np
from jax.experimental import pallas as pl
from jax.experimental.pallas import tpu as pltpu

_BASE = np.float32(1.0 - float(np.log(1 / (4 + 1) + 1.0)))


def _mask_draws(d, to_numpy):
    key = jax.random.key(1)
    k1, k2 = jax.random.split(key)
    drop_u = jax.random.uniform(k1, (d,), dtype=jnp.float32)
    pos = jax.random.randint(k2, (), 0, d)
    if to_numpy:
        drop_u = np.asarray(drop_u)
        adding = np.zeros((d,), np.float32)
        adding[int(pos)] = 1.0
    else:
        adding = jnp.zeros((d,), jnp.float32).at[pos].set(1.0)
    return drop_u.reshape(1, d), adding.reshape(1, d)


try:
    _CONST_128 = _mask_draws(128, to_numpy=True)
except Exception:
    _CONST_128 = None


_NBUF = 3


def _chunk_plan(n, bm):
    head = [96, bm - 96]
    tail = [bm - 96, 96]
    mid_rows = n - sum(head) - sum(tail)
    assert mid_rows % bm == 0
    sizes = head + [bm] * (mid_rows // bm) + tail
    plan, r0 = [], 0
    for sz in sizes:
        plan.append((r0, sz))
        r0 += sz
    return tuple(plan)


def _body(l_ref, drop_ref, add_ref, adj_ref, x_hbm, out_ref, *scratch,
          plan, n, d):
    bufs = scratch[:_NBUF]
    xv, o0, o1, sem, xsem, osem = scratch[_NBUF:]
    obs = (o0, o1)

    def cp(ci, slot):
        r0, sz = plan[ci]
        return pltpu.make_async_copy(
            adj_ref.at[pl.ds(r0, sz), :],
            bufs[slot].at[pl.ds(0, sz), :],
            sem.at[slot],
        )

    def ocp(ci, slot):
        r0, sz = plan[ci]
        return pltpu.make_async_copy(
            obs[slot].at[pl.ds(0, sz), :],
            out_ref.at[pl.ds(r0, sz), :],
            osem.at[slot],
        )

    xcp = pltpu.make_async_copy(x_hbm, xv, xsem)
    cp(0, 0).start()
    xcp.start()
    for ci in range(1, _NBUF):
        cp(ci, ci).start()
    rv = jnp.where(l_ref[0] <= 2, jnp.float32(0.0), _BASE)
    m = (drop_ref[...] < rv) | (add_ref[...] != 0.0)
    xcp.wait()
    nc = len(plan)
    for ci in range(nc):
        slot = ci % _NBUF
        cp(ci, slot).wait()
        r0, sz = plan[ci]
        h = jnp.dot(bufs[slot][0:sz, :], xv[...],
                    preferred_element_type=jnp.float32)
        oslot = ci % 2
        if ci >= 2:
            ocp(ci - 2, oslot).wait()
        obs[oslot][0:sz, :] = jnp.where(m, xv[r0 : r0 + sz, :], h)
        ocp(ci, oslot).start()
        if ci + _NBUF < nc:
            cp(ci + _NBUF, slot).start()
    ocp(nc - 2, (nc - 2) % 2).wait()
    ocp(nc - 1, (nc - 1) % 2).wait()


def kernel(input, adj, rate, l):
    n, d = input.shape
    if d == 128 and _CONST_128 is not None:
        drop_u, adding = _CONST_128
    else:
        drop_u, adding = _mask_draws(d, to_numpy=False)
    lv = jnp.asarray(l, jnp.int32).reshape(1)

    bm = 400
    plan = _chunk_plan(n, bm)
    return pl.pallas_call(
        functools.partial(_body, plan=plan, n=n, d=d),
        in_specs=[
            pl.BlockSpec(memory_space=pltpu.SMEM),
            pl.BlockSpec((1, d), lambda: (0, 0)),
            pl.BlockSpec((1, d), lambda: (0, 0)),
            pl.BlockSpec(memory_space=pl.ANY),
            pl.BlockSpec(memory_space=pl.ANY),
        ],
        out_specs=pl.BlockSpec(memory_space=pl.ANY),
        out_shape=jax.ShapeDtypeStruct((n, d), jnp.float32),
        scratch_shapes=(
            [pltpu.VMEM((bm, n), jnp.float32) for _ in range(_NBUF)]
            + [
                pltpu.VMEM((n, d), jnp.float32),
                pltpu.VMEM((bm, d), jnp.float32),
                pltpu.VMEM((bm, d), jnp.float32),
                pltpu.SemaphoreType.DMA((_NBUF,)),
                pltpu.SemaphoreType.DMA,
                pltpu.SemaphoreType.DMA((2,)),
            ]
        ),
    )(lv, jnp.asarray(drop_u), jnp.asarray(adding), adj, input)


# speedup vs baseline: 1.0227x; 1.0227x over previous
import functools

import jax
import jax.numpy as jnp
import numpy as np
from jax.experimental import pallas as pl
from jax.experimental.pallas import tpu as pltpu

_BASE = np.float32(1.0 - float(np.log(1 / (4 + 1) + 1.0)))


def _mask_draws(d, to_numpy):
    key = jax.random.key(1)
    k1, k2 = jax.random.split(key)
    drop_u = jax.random.uniform(k1, (d,), dtype=jnp.float32)
    pos = jax.random.randint(k2, (), 0, d)
    if to_numpy:
        drop_u = np.asarray(drop_u)
        adding = np.zeros((d,), np.float32)
        adding[int(pos)] = 1.0
    else:
        adding = jnp.zeros((d,), jnp.float32).at[pos].set(1.0)
    return drop_u.reshape(1, d), adding.reshape(1, d)


try:
    _CONST_128 = _mask_draws(128, to_numpy=True)
except Exception:
    _CONST_128 = None


_NBUF = 8


def _chunk_plan(n, bm):
    sizes = [bm] * (n // bm)
    if n % bm:
        sizes.append(n % bm)
    plan, r0 = [], 0
    for sz in sizes:
        plan.append((r0, sz))
        r0 += sz
    return tuple(plan)


def _body(l_ref, drop_ref, add_ref, adj_ref, x_hbm, out_ref, *scratch,
          plan, n, d):
    bufs = scratch[:_NBUF]
    xv, o0, o1, sem, xsem, osem = scratch[_NBUF:]
    obs = (o0, o1)

    def cp(ci, slot):
        r0, sz = plan[ci]
        return pltpu.make_async_copy(
            adj_ref.at[pl.ds(r0, sz), :],
            bufs[slot].at[pl.ds(0, sz), :],
            sem.at[slot],
        )

    def ocp(ci, slot):
        r0, sz = plan[ci]
        return pltpu.make_async_copy(
            obs[slot].at[pl.ds(0, sz), :],
            out_ref.at[pl.ds(r0, sz), :],
            osem.at[slot],
        )

    xcp = pltpu.make_async_copy(x_hbm, xv, xsem)
    cp(0, 0).start()
    xcp.start()
    for ci in range(1, _NBUF):
        cp(ci, ci).start()
    rv = jnp.where(l_ref[0] <= 2, jnp.float32(0.0), _BASE)
    m = (drop_ref[...] < rv) | (add_ref[...] != 0.0)
    xcp.wait()
    nc = len(plan)
    for ci in range(nc):
        slot = ci % _NBUF
        cp(ci, slot).wait()
        r0, sz = plan[ci]
        h = jnp.dot(bufs[slot][0:sz, :], xv[...],
                    preferred_element_type=jnp.float32)
        oslot = ci % 2
        if ci >= 2:
            ocp(ci - 2, oslot).wait()
        obs[oslot][0:sz, :] = jnp.where(m, xv[r0 : r0 + sz, :], h)
        ocp(ci, oslot).start()
        if ci + _NBUF < nc:
            cp(ci + _NBUF, slot).start()
    ocp(nc - 2, (nc - 2) % 2).wait()
    ocp(nc - 1, (nc - 1) % 2).wait()


def kernel(input, adj, rate, l):
    n, d = input.shape
    if d == 128 and _CONST_128 is not None:
        drop_u, adding = _CONST_128
    else:
        drop_u, adding = _mask_draws(d, to_numpy=False)
    lv = jnp.asarray(l, jnp.int32).reshape(1)

    bm = 96
    plan = _chunk_plan(n, bm)
    return pl.pallas_call(
        functools.partial(_body, plan=plan, n=n, d=d),
        in_specs=[
            pl.BlockSpec(memory_space=pltpu.SMEM),
            pl.BlockSpec((1, d), lambda: (0, 0)),
            pl.BlockSpec((1, d), lambda: (0, 0)),
            pl.BlockSpec(memory_space=pl.ANY),
            pl.BlockSpec(memory_space=pl.ANY),
        ],
        out_specs=pl.BlockSpec(memory_space=pl.ANY),
        out_shape=jax.ShapeDtypeStruct((n, d), jnp.float32),
        scratch_shapes=(
            [pltpu.VMEM((bm, n), jnp.float32) for _ in range(_NBUF)]
            + [
                pltpu.VMEM((n, d), jnp.float32),
                pltpu.VMEM((bm, d), jnp.float32),
                pltpu.VMEM((bm, d), jnp.float32),
                pltpu.SemaphoreType.DMA((_NBUF,)),
                pltpu.SemaphoreType.DMA,
                pltpu.SemaphoreType.DMA((2,)),
            ]
        ),
    )(lv, jnp.asarray(drop_u), jnp.asarray(adding), adj, input)


# speedup vs baseline: 1.0234x; 1.0007x over previous
import functools

import jax
import jax.numpy as jnp
import numpy as np
from jax.experimental import pallas as pl
from jax.experimental.pallas import tpu as pltpu

_BASE = np.float32(1.0 - float(np.log(1 / (4 + 1) + 1.0)))


def _mask_draws(d, to_numpy):
    key = jax.random.key(1)
    k1, k2 = jax.random.split(key)
    drop_u = jax.random.uniform(k1, (d,), dtype=jnp.float32)
    pos = jax.random.randint(k2, (), 0, d)
    if to_numpy:
        drop_u = np.asarray(drop_u)
        adding = np.zeros((d,), np.float32)
        adding[int(pos)] = 1.0
    else:
        adding = jnp.zeros((d,), jnp.float32).at[pos].set(1.0)
    return drop_u.reshape(1, d), adding.reshape(1, d)


try:
    _CONST_128 = _mask_draws(128, to_numpy=True)
except Exception:
    _CONST_128 = None


def _body(l_ref, drop_ref, add_ref, adj_ref, x_ref, out_ref, *, bm):
    i = pl.program_id(0)
    rv = jnp.where(l_ref[0] <= 2, jnp.float32(0.0), _BASE)
    m = (drop_ref[...] < rv) | (add_ref[...] != 0.0)
    h = jnp.dot(adj_ref[...], x_ref[...], preferred_element_type=jnp.float32)
    xrow = x_ref[pl.ds(i * bm, bm), :]
    out_ref[...] = jnp.where(m, xrow, h)


def kernel(input, adj, rate, l):
    n, d = input.shape
    if d == 128 and _CONST_128 is not None:
        drop_u, adding = _CONST_128
    else:
        drop_u, adding = _mask_draws(d, to_numpy=False)
    lv = jnp.asarray(l, jnp.int32).reshape(1)

    bm = 400
    grid = (n // bm,)
    return pl.pallas_call(
        functools.partial(_body, bm=bm),
        grid=grid,
        in_specs=[
            pl.BlockSpec(memory_space=pltpu.SMEM),
            pl.BlockSpec((1, d), lambda m: (0, 0)),
            pl.BlockSpec((1, d), lambda m: (0, 0)),
            pl.BlockSpec((bm, n), lambda m: (m, 0)),
            pl.BlockSpec((n, d), lambda m: (0, 0)),
        ],
        out_specs=pl.BlockSpec((bm, d), lambda m: (m, 0)),
        out_shape=jax.ShapeDtypeStruct((n, d), jnp.float32),
        compiler_params=pltpu.CompilerParams(
            dimension_semantics=("parallel",),
        ),
    )(lv, jnp.asarray(drop_u), jnp.asarray(adding), adj, input)
